# baseline (speedup 1.0000x reference)
import jax
import jax.numpy as jnp
from jax import lax
from jax.experimental import pallas as pl
from jax.experimental.pallas import tpu as pltpu
from jax.experimental.pallas import tpu_sc as plsc

BATCH = 16384
EMBED = 64
NTOK = 1000

NUM_CORES = 2
NUM_SUBCORES = 16
NW = NUM_CORES * NUM_SUBCORES
B_PER_W = BATCH // NW
IDX_CHUNK = 128
N_STREAMS = B_PER_W // IDX_CHUNK


def _gather_body(idx_hbm, table_hbm, out_hbm, idx_v, rows_v, sem):
    wid = lax.axis_index("s") * NUM_CORES + lax.axis_index("c")
    base = wid * B_PER_W
    pltpu.sync_copy(idx_hbm.at[pl.ds(wid * N_STREAMS, N_STREAMS)], idx_v)
    copies = [
        pltpu.async_copy(
            table_hbm.at[idx_v.at[j]],
            rows_v.at[pl.ds(j * IDX_CHUNK, IDX_CHUNK)],
            sem,
        )
        for j in range(N_STREAMS)
    ]
    for c in copies:
        c.wait()
    pltpu.sync_copy(rows_v, out_hbm.at[pl.ds(base, B_PER_W)])


_gather = pl.kernel(
    _gather_body,
    out_type=jax.ShapeDtypeStruct((BATCH, EMBED), jnp.float32),
    mesh=plsc.VectorSubcoreMesh(core_axis_name="c", subcore_axis_name="s"),
    scratch_types=[
        pltpu.VMEM((N_STREAMS, IDX_CHUNK), jnp.int32),
        pltpu.VMEM((B_PER_W, EMBED), jnp.float32),
        pltpu.SemaphoreType.DMA,
    ],
    compiler_params=pltpu.CompilerParams(use_tc_tiling_on_sc=False),
)

BM = 512


def _decode_body(h_ref, d_ref, o_ref):
    o_ref[...] = jnp.dot(h_ref[...], d_ref[...],
                         preferred_element_type=jnp.float32)


_decode = pl.pallas_call(
    _decode_body,
    grid=(BATCH // BM,),
    in_specs=[
        pl.BlockSpec((BM, EMBED), lambda i: (i, 0)),
        pl.BlockSpec((EMBED, NTOK), lambda i: (0, 0)),
    ],
    out_specs=pl.BlockSpec((BM, NTOK), lambda i: (i, 0)),
    out_shape=jax.ShapeDtypeStruct((BATCH, NTOK), jnp.float32),
    compiler_params=pltpu.CompilerParams(
        dimension_semantics=("parallel",),
    ),
)


def kernel(x, embedding_weight, decoder_weight):
    idx2d = x.astype(jnp.int32).reshape(NW * N_STREAMS, IDX_CHUNK)
    hidden = _gather(idx2d, embedding_weight)
    return _decode(hidden, decoder_weight.T)


# speedup vs baseline: 1.9951x; 1.9951x over previous
import jax
import jax.numpy as jnp
from jax import lax
from jax.experimental import pallas as pl
from jax.experimental.pallas import tpu as pltpu
from jax.experimental.pallas import tpu_sc as plsc

BATCH = 16384
EMBED = 64
NTOK = 1000

NUM_CORES = 2
NUM_SUBCORES = 16
NW = NUM_CORES * NUM_SUBCORES
B_PER_W = BATCH // NW
LANES = 128
WAVE = 4


def _gather_body(idx_hbm, tableT_hbm, outT_hbm, idx_v, slabs, rows_v,
                 sem_a, sem_b):
    wid = lax.axis_index("s") * NUM_CORES + lax.axis_index("c")
    base = pl.multiple_of(wid * B_PER_W, B_PER_W)
    pltpu.sync_copy(idx_hbm.at[pl.ds(base, B_PER_W)], idx_v)

    rows16 = [lax.iota(jnp.int32, 16) + 16 * q for q in range(EMBED // 16)]

    def fire(ss, buf, sem):
        for k in range(WAVE):
            c = pl.multiple_of((ss[k] >> 7) << 7, LANES)
            pltpu.async_copy(
                tableT_hbm.at[:, pl.ds(c, LANES)], slabs.at[buf, k], sem)

    def wait_wave(buf, sem):
        for k in range(WAVE):
            pltpu.make_async_copy(
                tableT_hbm.at[:, pl.ds(0, LANES)], slabs.at[buf, k], sem
            ).wait()

    def select(ss, j, buf):
        for k in range(WAVE):
            lane = jnp.full((16,), ss[k] & (LANES - 1), jnp.int32)
            col = jnp.full((16,), j + k, jnp.int32)
            for q in range(EMBED // 16):
                vals = plsc.load_gather(slabs.at[buf, k], [rows16[q], lane])
                plsc.store_scatter(rows_v, [rows16[q], col], vals)

    def body(t, _):
        j0 = pl.multiple_of(t * 4 * WAVE, 4 * WAVE)
        vec = idx_v[pl.ds(j0, 16)]
        ss = [vec[k] for k in range(16)]
        w = [ss[WAVE * i:WAVE * (i + 1)] for i in range(4)]
        fire(w[0], 0, sem_a)
        fire(w[1], 1, sem_b)
        wait_wave(0, sem_a)
        select(w[0], j0, 0)
        fire(w[2], 0, sem_a)
        wait_wave(1, sem_b)
        select(w[1], j0 + WAVE, 1)
        fire(w[3], 1, sem_b)
        wait_wave(0, sem_a)
        select(w[2], j0 + 2 * WAVE, 0)
        wait_wave(1, sem_b)
        select(w[3], j0 + 3 * WAVE, 1)
        return ()

    lax.fori_loop(0, B_PER_W // (4 * WAVE), body, ())
    pltpu.sync_copy(rows_v, outT_hbm.at[:, pl.ds(base, B_PER_W)])


_gather = pl.kernel(
    _gather_body,
    out_type=jax.ShapeDtypeStruct((EMBED, BATCH), jnp.float32),
    mesh=plsc.VectorSubcoreMesh(core_axis_name="c", subcore_axis_name="s"),
    scratch_types=[
        pltpu.VMEM((B_PER_W,), jnp.int32),
        pltpu.VMEM((2, WAVE, EMBED, LANES), jnp.float32),
        pltpu.VMEM((EMBED, B_PER_W), jnp.float32),
        pltpu.SemaphoreType.DMA,
        pltpu.SemaphoreType.DMA,
    ],
    compiler_params=pltpu.CompilerParams(needs_layout_passes=False),
)

BM = 512


def _decode_body(ht_ref, d_ref, o_ref):
    o_ref[...] = jax.lax.dot_general(
        ht_ref[...], d_ref[...],
        dimension_numbers=(((0,), (0,)), ((), ())),
        preferred_element_type=jnp.float32,
    )


_decode = pl.pallas_call(
    _decode_body,
    grid=(BATCH // BM,),
    in_specs=[
        pl.BlockSpec((EMBED, BM), lambda i: (0, i)),
        pl.BlockSpec((EMBED, NTOK), lambda i: (0, 0)),
    ],
    out_specs=pl.BlockSpec((BM, NTOK), lambda i: (i, 0)),
    out_shape=jax.ShapeDtypeStruct((BATCH, NTOK), jnp.float32),
    compiler_params=pltpu.CompilerParams(
        dimension_semantics=("parallel",),
    ),
)


def kernel(x, embedding_weight, decoder_weight):
    hiddenT = _gather(x.astype(jnp.int32), embedding_weight.T)
    return _decode(hiddenT, decoder_weight.T)


# speedup vs baseline: 2.0686x; 1.0368x over previous
import jax
import jax.numpy as jnp
from jax import lax
from jax.experimental import pallas as pl
from jax.experimental.pallas import tpu as pltpu
from jax.experimental.pallas import tpu_sc as plsc

BATCH = 16384
EMBED = 64
NTOK = 1000

NUM_CORES = 2
NUM_SUBCORES = 16
NW = NUM_CORES * NUM_SUBCORES
B_PER_W = BATCH // NW
LANES = 128
GRAN = 128
WAVE = 4


def _gather_body(idx_hbm, tableT_hbm, outT_hbm, idx_v, slabs, rows_v,
                 sem_a, sem_b):
    wid = lax.axis_index("s") * NUM_CORES + lax.axis_index("c")
    base = pl.multiple_of(wid * B_PER_W, B_PER_W)
    pltpu.sync_copy(idx_hbm.at[pl.ds(base, B_PER_W)], idx_v)

    rows16 = [lax.iota(jnp.int32, 16) + 16 * q for q in range(EMBED // 16)]

    def fire(ss, buf, sem):
        for k in range(WAVE):
            c = pl.multiple_of((ss[k] >> 7) << 7, LANES)
            pltpu.async_copy(
                tableT_hbm.at[:, pl.ds(c, GRAN)], slabs.at[buf, k], sem)

    def wait_wave(buf, sem):
        for k in range(WAVE):
            pltpu.make_async_copy(
                tableT_hbm.at[:, pl.ds(0, GRAN)], slabs.at[buf, k], sem
            ).wait()

    def select(ss, j, buf):
        for k in range(WAVE):
            lane = jnp.full((16,), ss[k] & (GRAN - 1), jnp.int32)
            col = jnp.full((16,), j + k, jnp.int32)
            for q in range(EMBED // 16):
                vals = plsc.load_gather(slabs.at[buf, k], [rows16[q], lane])
                plsc.store_scatter(rows_v, [rows16[q], col], vals)

    def body(t, _):
        j0 = pl.multiple_of(t * 4 * WAVE, 4 * WAVE)
        vec = idx_v[pl.ds(j0, 16)]
        ss = [vec[k] for k in range(16)]
        w = [ss[WAVE * i:WAVE * (i + 1)] for i in range(4)]
        fire(w[0], 0, sem_a)
        fire(w[1], 1, sem_b)
        wait_wave(0, sem_a)
        select(w[0], j0, 0)
        fire(w[2], 0, sem_a)
        wait_wave(1, sem_b)
        select(w[1], j0 + WAVE, 1)
        fire(w[3], 1, sem_b)
        wait_wave(0, sem_a)
        select(w[2], j0 + 2 * WAVE, 0)
        wait_wave(1, sem_b)
        select(w[3], j0 + 3 * WAVE, 1)
        return ()

    lax.fori_loop(0, B_PER_W // (4 * WAVE), body, ())
    pltpu.sync_copy(rows_v, outT_hbm.at[:, pl.ds(base, B_PER_W)])


_gather = pl.kernel(
    _gather_body,
    out_type=jax.ShapeDtypeStruct((EMBED, BATCH), jnp.float32),
    mesh=plsc.VectorSubcoreMesh(core_axis_name="c", subcore_axis_name="s"),
    scratch_types=[
        pltpu.VMEM((B_PER_W,), jnp.int32),
        pltpu.VMEM((2, WAVE, EMBED, GRAN), jnp.float32),
        pltpu.VMEM((EMBED, B_PER_W), jnp.float32),
        pltpu.SemaphoreType.DMA,
        pltpu.SemaphoreType.DMA,
    ],
    compiler_params=pltpu.CompilerParams(needs_layout_passes=False),
)

BM = 2048


def _decode_body(ht_ref, d_ref, o_ref):
    o_ref[...] = jax.lax.dot_general(
        ht_ref[...], d_ref[...],
        dimension_numbers=(((0,), (0,)), ((), ())),
        preferred_element_type=jnp.float32,
    )


_decode = pl.pallas_call(
    _decode_body,
    grid=(BATCH // BM,),
    in_specs=[
        pl.BlockSpec((EMBED, BM), lambda i: (0, i)),
        pl.BlockSpec((EMBED, NTOK), lambda i: (0, 0)),
    ],
    out_specs=pl.BlockSpec((BM, NTOK), lambda i: (i, 0)),
    out_shape=jax.ShapeDtypeStruct((BATCH, NTOK), jnp.float32),
    compiler_params=pltpu.CompilerParams(
        dimension_semantics=("parallel",),
    ),
)


def kernel(x, embedding_weight, decoder_weight):
    hiddenT = _gather(x.astype(jnp.int32), embedding_weight.T)
    return _decode(hiddenT, decoder_weight.T)
